# Initial kernel scaffold; baseline (speedup 1.0000x reference)
import jax
import jax.numpy as jnp
from jax.experimental import pallas as pl


def kernel(token_ids, embed_weight, scale):
    raise NotImplementedError("write your pallas kernel here")


# speedup vs baseline: 1.0591x; 1.0591x over previous
import functools

import jax
import jax.numpy as jnp
from jax import lax
from jax.experimental import pallas as pl
from jax.experimental.pallas import tpu as pltpu
from jax.experimental.pallas import tpu_sc as plsc

_VOCAB_SIZE = 1000000
_MOD = _VOCAB_SIZE - 1
_L = 16

_NUM_CORES = 2
_NUM_SUBCORES = 16
_NW = _NUM_CORES * _NUM_SUBCORES

_CHUNK = 128


@functools.partial(jax.jit, static_argnames=("seq", "dim"))
def _sc_embed(prev, curr, table, scale16, *, seq, dim):
    n = curr.shape[0]
    per_w = n // _NW
    nchunk = per_w // _CHUNK
    vperchunk = _CHUNK // _L

    mesh = plsc.VectorSubcoreMesh(
        core_axis_name="c", subcore_axis_name="s",
        num_cores=_NUM_CORES, num_subcores=_NUM_SUBCORES)

    @functools.partial(
        pl.kernel,
        out_type=jax.ShapeDtypeStruct((n, dim), jnp.float32),
        mesh=mesh,
        scratch_types=[
            pltpu.VMEM((per_w,), jnp.int32),
            pltpu.VMEM((per_w,), jnp.int32),
            pltpu.VMEM((nchunk, _CHUNK), jnp.int32),
            pltpu.VMEM((2, _CHUNK, dim), jnp.float32),
            pltpu.VMEM((_L,), jnp.float32),
            pltpu.SemaphoreType.DMA,
            pltpu.SemaphoreType.DMA,
            pltpu.SemaphoreType.DMA,
            pltpu.SemaphoreType.DMA,
        ],
    )
    def k(prev_h, curr_h, table_h, scale_h, out_h,
          pv, cv, idx, rows, sv_ref, g0, g1, o0, o1):
        wid = lax.axis_index("s") * _NUM_CORES + lax.axis_index("c")
        base = wid * per_w

        pltpu.sync_copy(prev_h.at[pl.ds(base, per_w)], pv)
        pltpu.sync_copy(curr_h.at[pl.ds(base, per_w)], cv)
        pltpu.sync_copy(scale_h, sv_ref)
        sv = sv_ref[...]

        gsem = (g0, g1)
        osem = (o0, o1)
        gather_d = [None] * nchunk
        out_d = [None] * nchunk
        lane = lax.iota(jnp.int32, _L)

        for c in range(nchunk + 1):
            if c < nchunk:
                buf = c & 1
                for kk in range(vperchunk):
                    off = (c * vperchunk + kk) * _L
                    tp = pv[pl.ds(off, _L)]
                    tc = cv[pl.ds(off, _L)]
                    h = (jnp.int32(36313) * tc) ^ (jnp.int32(27191) * tp)
                    h = h % jnp.int32(_MOD)
                    pos = base + jnp.int32(off) + lane
                    h = jnp.where((pos & jnp.int32(seq - 1)) == 0,
                                  jnp.int32(_MOD), h)
                    idx[c, pl.ds(kk * _L, _L)] = h
                if c >= 2:
                    out_d[c - 2].wait()
                gather_d[c] = pltpu.async_copy(
                    table_h.at[idx.at[c]], rows.at[buf], gsem[buf])
            if c >= 1:
                p = c - 1
                pb = p & 1
                gather_d[p].wait()

                def body(r, _, pb=pb):
                    for kk in range(dim // _L):
                        s = pl.ds(kk * _L, _L)
                        rows[pb, r, s] = rows[pb, r, s] * sv
                    return 0

                lax.fori_loop(0, _CHUNK, body, 0)
                out_d[p] = pltpu.async_copy(
                    rows.at[pb],
                    out_h.at[pl.ds(base + p * _CHUNK, _CHUNK)],
                    osem[pb])
        out_d[nchunk - 2].wait()
        out_d[nchunk - 1].wait()

    return k(prev, curr, table, scale16)


def kernel(token_ids, embed_weight, scale):
    b, seq = token_ids.shape
    dim = embed_weight.shape[1]
    flat = token_ids.reshape(-1).astype(jnp.int32)
    prev = jnp.concatenate([jnp.zeros((1,), jnp.int32), flat[:-1]])
    scale16 = jnp.broadcast_to(scale.astype(jnp.float32), (_L,))
    out = _sc_embed(prev, flat, embed_weight, scale16, seq=seq, dim=dim)
    return out.reshape(b, seq, dim)


# speedup vs baseline: 1.2434x; 1.1740x over previous
import functools

import jax
import jax.numpy as jnp
from jax import lax
from jax.experimental import pallas as pl
from jax.experimental.pallas import tpu as pltpu
from jax.experimental.pallas import tpu_sc as plsc

_VOCAB_SIZE = 1000000
_MOD = _VOCAB_SIZE - 1
_L = 16

_NUM_CORES = 2
_NUM_SUBCORES = 16
_NW = _NUM_CORES * _NUM_SUBCORES

_CHUNK = 128


@functools.partial(jax.jit, static_argnames=("seq", "dim"))
def _sc_embed(prev, curr, table, scale16, *, seq, dim):
    n = curr.shape[0]
    b = n // seq
    per_w = n // _NW
    nchunk = per_w // _CHUNK
    vperchunk = _CHUNK // _L

    mesh = plsc.VectorSubcoreMesh(
        core_axis_name="c", subcore_axis_name="s",
        num_cores=_NUM_CORES, num_subcores=_NUM_SUBCORES)

    @functools.partial(
        pl.kernel,
        out_type=jax.ShapeDtypeStruct((n, dim), jnp.float32),
        mesh=mesh,
        scratch_types=[
            pltpu.VMEM((per_w,), jnp.int32),
            pltpu.VMEM((per_w,), jnp.int32),
            pltpu.VMEM((nchunk, _CHUNK), jnp.int32),
            pltpu.VMEM((2, _CHUNK, dim), jnp.float32),
            pltpu.VMEM((_L,), jnp.float32),
            pltpu.SemaphoreType.DMA,
            pltpu.SemaphoreType.DMA,
            pltpu.SemaphoreType.DMA,
            pltpu.SemaphoreType.DMA,
        ],
    )
    def k(prev_h, curr_h, table_h, scale_h, out_h,
          pv, cv, idx, rows, sv_ref, g0, g1, o0, o1):
        wid = lax.axis_index("s") * _NUM_CORES + lax.axis_index("c")
        base = wid * per_w
        wpb = seq // per_w
        bi = lax.div(wid, wpb)
        off = lax.rem(wid, wpb) * per_w

        pltpu.sync_copy(prev_h.at[pl.ds(base, per_w)], pv)
        pltpu.sync_copy(curr_h.at[pl.ds(base, per_w)], cv)
        pltpu.sync_copy(scale_h, sv_ref)
        sv = sv_ref[...]

        gsem = (g0, g1)
        osem = (o0, o1)
        gather_d = [None] * nchunk
        out_d = [None] * nchunk
        lane = lax.iota(jnp.int32, _L)

        def hash_body(i, _):
            off = i * _L
            tp = pv[pl.ds(off, _L)]
            tc = cv[pl.ds(off, _L)]
            h = (jnp.int32(36313) * tc) ^ (jnp.int32(27191) * tp)
            h = h % jnp.int32(_MOD)
            pos = base + off + lane
            h = jnp.where((pos & jnp.int32(seq - 1)) == 0, jnp.int32(_MOD), h)
            c = lax.shift_right_logical(i, 3)
            idx[c, pl.ds((i & (vperchunk - 1)) * _L, _L)] = h
            return 0

        lax.fori_loop(0, per_w // _L, hash_body, 0)

        for c in range(nchunk + 1):
            if c < nchunk:
                buf = c & 1
                if c >= 2:
                    out_d[c - 2].wait()
                gather_d[c] = pltpu.async_copy(
                    table_h.at[idx.at[c]], rows.at[buf], gsem[buf])
            if c >= 1:
                p = c - 1
                pb = p & 1
                gather_d[p].wait()

                def body(r, _, pb=pb):
                    for kk in range(dim // _L):
                        s = pl.ds(kk * _L, _L)
                        rows[pb, r, s] = rows[pb, r, s] * sv
                    return 0

                lax.fori_loop(0, _CHUNK, body, 0)
                out_d[p] = pltpu.async_copy(
                    rows.at[pb],
                    out_h.at[pl.ds(base + p * _CHUNK, _CHUNK)],
                    osem[pb])
        out_d[nchunk - 2].wait()
        out_d[nchunk - 1].wait()

    return k(prev, curr, table, scale16)


def kernel(token_ids, embed_weight, scale):
    b, seq = token_ids.shape
    dim = embed_weight.shape[1]
    flat = token_ids.reshape(-1).astype(jnp.int32)
    prev = jnp.concatenate([jnp.zeros((1,), jnp.int32), flat[:-1]])
    scale16 = jnp.broadcast_to(scale.astype(jnp.float32), (_L,))
    out = _sc_embed(prev, flat, embed_weight, scale16, seq=seq, dim=dim)
    return out.reshape(b, seq, dim)


# speedup vs baseline: 1.3008x; 1.0462x over previous
import functools

import jax
import jax.numpy as jnp
from jax import lax
from jax.experimental import pallas as pl
from jax.experimental.pallas import tpu as pltpu
from jax.experimental.pallas import tpu_sc as plsc

_VOCAB_SIZE = 1000000
_MOD = _VOCAB_SIZE - 1
_L = 16

_NUM_CORES = 2
_NUM_SUBCORES = 16
_NW = _NUM_CORES * _NUM_SUBCORES

_CHUNK = 128


@functools.partial(jax.jit, static_argnames=("seq", "dim"))
def _sc_embed(prev, curr, table, scale16, *, seq, dim):
    n = curr.shape[0]
    b = n // seq
    per_w = n // _NW
    nchunk = per_w // _CHUNK
    vperchunk = _CHUNK // _L

    mesh = plsc.VectorSubcoreMesh(
        core_axis_name="c", subcore_axis_name="s",
        num_cores=_NUM_CORES, num_subcores=_NUM_SUBCORES)

    @functools.partial(
        pl.kernel,
        out_type=jax.ShapeDtypeStruct((n, dim), jnp.float32),
        mesh=mesh,
        scratch_types=[
            pltpu.VMEM((per_w,), jnp.int32),
            pltpu.VMEM((per_w,), jnp.int32),
            pltpu.VMEM((nchunk, _CHUNK), jnp.int32),
            pltpu.VMEM((3, _CHUNK, dim), jnp.float32),
            pltpu.VMEM((_L,), jnp.float32),
            pltpu.SemaphoreType.DMA,
            pltpu.SemaphoreType.DMA,
            pltpu.SemaphoreType.DMA,
            pltpu.SemaphoreType.DMA,
            pltpu.SemaphoreType.DMA,
            pltpu.SemaphoreType.DMA,
        ],
    )
    def k(prev_h, curr_h, table_h, scale_h, out_h,
          pv, cv, idx, rows, sv_ref, g0, g1, g2, o0, o1, o2):
        wid = lax.axis_index("s") * _NUM_CORES + lax.axis_index("c")
        base = wid * per_w
        wpb = seq // per_w
        bi = lax.div(wid, wpb)
        off = lax.rem(wid, wpb) * per_w

        pltpu.sync_copy(prev_h.at[pl.ds(base, per_w)], pv)
        pltpu.sync_copy(curr_h.at[pl.ds(base, per_w)], cv)
        pltpu.sync_copy(scale_h, sv_ref)
        sv = sv_ref[...]

        nbuf = 3
        gsem = (g0, g1, g2)
        osem = (o0, o1, o2)
        gather_d = [None] * nchunk
        out_d = [None] * nchunk
        lane = lax.iota(jnp.int32, _L)

        def hash_body(i, _):
            off = i * _L
            tp = pv[pl.ds(off, _L)]
            tc = cv[pl.ds(off, _L)]
            h = (jnp.int32(36313) * tc) ^ (jnp.int32(27191) * tp)
            h = h % jnp.int32(_MOD)
            pos = base + off + lane
            h = jnp.where((pos & jnp.int32(seq - 1)) == 0, jnp.int32(_MOD), h)
            c = lax.shift_right_logical(i, 3)
            idx[c, pl.ds((i & (vperchunk - 1)) * _L, _L)] = h
            return 0

        lax.fori_loop(0, vperchunk, hash_body, 0)
        gather_d[0] = pltpu.async_copy(
            table_h.at[idx.at[0]], rows.at[0], gsem[0])
        lax.fori_loop(vperchunk, per_w // _L, hash_body, 0)

        for c in range(nchunk + 1):
            if 0 < c < nchunk:
                buf = c % nbuf
                if c >= nbuf:
                    out_d[c - nbuf].wait()
                gather_d[c] = pltpu.async_copy(
                    table_h.at[idx.at[c]], rows.at[buf], gsem[buf])
            if c >= 1:
                p = c - 1
                pb = p % nbuf
                gather_d[p].wait()

                def body(r2, _, pb=pb):
                    r = r2 * 2
                    for kk in range(2 * dim // _L):
                        s = pl.ds((kk & (dim // _L - 1)) * _L, _L)
                        rr = r + (kk * _L) // dim
                        rows[pb, rr, s] = rows[pb, rr, s] * sv
                    return 0

                lax.fori_loop(0, _CHUNK // 2, body, 0)
                out_d[p] = pltpu.async_copy(
                    rows.at[pb],
                    out_h.at[pl.ds(base + p * _CHUNK, _CHUNK)],
                    osem[pb])
        for p in range(nchunk - nbuf, nchunk):
            if p >= 0:
                out_d[p].wait()

    return k(prev, curr, table, scale16)


def kernel(token_ids, embed_weight, scale):
    b, seq = token_ids.shape
    dim = embed_weight.shape[1]
    flat = token_ids.reshape(-1).astype(jnp.int32)
    prev = jnp.concatenate([jnp.zeros((1,), jnp.int32), flat[:-1]])
    scale16 = jnp.broadcast_to(scale.astype(jnp.float32), (_L,))
    out = _sc_embed(prev, flat, embed_weight, scale16, seq=seq, dim=dim)
    return out.reshape(b, seq, dim)


# speedup vs baseline: 1.3053x; 1.0034x over previous
import functools

import jax
import jax.numpy as jnp
from jax import lax
from jax.experimental import pallas as pl
from jax.experimental.pallas import tpu as pltpu
from jax.experimental.pallas import tpu_sc as plsc

_VOCAB_SIZE = 1000000
_MOD = _VOCAB_SIZE - 1
_L = 16

_NUM_CORES = 2
_NUM_SUBCORES = 16
_NW = _NUM_CORES * _NUM_SUBCORES

_CHUNK = 128
_PAD = 8


@functools.partial(jax.jit, static_argnames=("seq", "dim"))
def _sc_embed(curr, table, scale16, *, seq, dim):
    n = curr.shape[0]
    per_w = n // _NW
    nchunk = per_w // _CHUNK
    vperchunk = _CHUNK // _L

    mesh = plsc.VectorSubcoreMesh(
        core_axis_name="c", subcore_axis_name="s",
        num_cores=_NUM_CORES, num_subcores=_NUM_SUBCORES)

    @functools.partial(
        pl.kernel,
        out_type=jax.ShapeDtypeStruct((n, dim), jnp.float32),
        mesh=mesh,
        scratch_types=[
            pltpu.VMEM((per_w + 2 * _PAD,), jnp.int32),
            pltpu.VMEM((nchunk, _CHUNK), jnp.int32),
            pltpu.VMEM((3, _CHUNK, dim), jnp.float32),
            pltpu.VMEM((_L,), jnp.float32),
            pltpu.SemaphoreType.DMA,
            pltpu.SemaphoreType.DMA,
            pltpu.SemaphoreType.DMA,
            pltpu.SemaphoreType.DMA,
            pltpu.SemaphoreType.DMA,
            pltpu.SemaphoreType.DMA,
        ],
    )
    def k(curr_h, table_h, scale_h, out_h,
          tok, idx, rows, sv_ref, g0, g1, g2, o0, o1, o2):
        wid = lax.axis_index("s") * _NUM_CORES + lax.axis_index("c")
        base = wid * per_w

        @pl.when(wid == 0)
        def _():
            pltpu.sync_copy(curr_h.at[pl.ds(0, per_w + _PAD)],
                            tok.at[pl.ds(_PAD, per_w + _PAD)])

        @pl.when(wid != 0)
        def _():
            pltpu.sync_copy(curr_h.at[pl.ds(base - _PAD, per_w + 2 * _PAD)],
                            tok)

        pltpu.sync_copy(scale_h, sv_ref)
        sv = sv_ref[...]

        nbuf = 3
        gsem = (g0, g1, g2)
        osem = (o0, o1, o2)
        gather_d = [None] * nchunk
        out_d = [None] * nchunk
        lane = lax.iota(jnp.int32, _L)

        def hash_body(i, _):
            off = i * _L
            tp = tok[pl.ds(off + _PAD - 1, _L)]
            tc = tok[pl.ds(off + _PAD, _L)]
            h = (jnp.int32(36313) * tc) ^ (jnp.int32(27191) * tp)
            h = h % jnp.int32(_MOD)
            pos = base + off + lane
            h = jnp.where((pos & jnp.int32(seq - 1)) == 0, jnp.int32(_MOD), h)
            c = lax.shift_right_logical(i, 3)
            idx[c, pl.ds((i & (vperchunk - 1)) * _L, _L)] = h
            return 0

        lax.fori_loop(0, vperchunk, hash_body, 0)
        gather_d[0] = pltpu.async_copy(
            table_h.at[idx.at[0]], rows.at[0], gsem[0])
        lax.fori_loop(vperchunk, per_w // _L, hash_body, 0)

        for c in range(nchunk + 1):
            if 0 < c < nchunk:
                buf = c % nbuf
                if c >= nbuf:
                    out_d[c - nbuf].wait()
                gather_d[c] = pltpu.async_copy(
                    table_h.at[idx.at[c]], rows.at[buf], gsem[buf])
            if c >= 1:
                p = c - 1
                pb = p % nbuf
                gather_d[p].wait()

                def body(r4, _, pb=pb):
                    r = r4 * 4
                    for kk in range(4 * dim // _L):
                        s = pl.ds((kk & (dim // _L - 1)) * _L, _L)
                        rr = r + (kk * _L) // dim
                        rows[pb, rr, s] = rows[pb, rr, s] * sv
                    return 0

                lax.fori_loop(0, _CHUNK // 4, body, 0)
                out_d[p] = pltpu.async_copy(
                    rows.at[pb],
                    out_h.at[pl.ds(base + p * _CHUNK, _CHUNK)],
                    osem[pb])
        for p in range(nchunk - nbuf, nchunk):
            if p >= 0:
                out_d[p].wait()

    return k(curr, table, scale16)


def kernel(token_ids, embed_weight, scale):
    b, seq = token_ids.shape
    dim = embed_weight.shape[1]
    flat = token_ids.reshape(-1).astype(jnp.int32)
    scale16 = jnp.broadcast_to(scale.astype(jnp.float32), (_L,))
    out = _sc_embed(flat, embed_weight, scale16, seq=seq, dim=dim)
    return out.reshape(b, seq, dim)


# speedup vs baseline: 1.3519x; 1.0357x over previous
import functools

import jax
import jax.numpy as jnp
from jax import lax
from jax.experimental import pallas as pl
from jax.experimental.pallas import tpu as pltpu
from jax.experimental.pallas import tpu_sc as plsc

_VOCAB_SIZE = 1000000
_MOD = _VOCAB_SIZE - 1
_L = 16

_NUM_CORES = 2
_NUM_SUBCORES = 16
_NW = _NUM_CORES * _NUM_SUBCORES

_CHUNK = 128
_PAD = 8


@functools.partial(jax.jit, static_argnames=("seq", "dim"))
def _sc_embed(curr, table, scale16, *, seq, dim):
    n = curr.shape[0]
    per_w = n // _NW
    nchunk = per_w // _CHUNK
    vperchunk = _CHUNK // _L

    mesh = plsc.VectorSubcoreMesh(
        core_axis_name="c", subcore_axis_name="s",
        num_cores=_NUM_CORES, num_subcores=_NUM_SUBCORES)

    @functools.partial(
        pl.kernel,
        out_type=jax.ShapeDtypeStruct((n, dim), jnp.float32),
        mesh=mesh,
        scratch_types=[
            pltpu.VMEM((per_w + 2 * _PAD,), jnp.int32),
            pltpu.VMEM((nchunk, _CHUNK), jnp.int32),
            pltpu.VMEM((4, _CHUNK, dim), jnp.float32),
            pltpu.VMEM((_L,), jnp.float32),
            pltpu.SemaphoreType.DMA,
            pltpu.SemaphoreType.DMA,
            pltpu.SemaphoreType.DMA,
            pltpu.SemaphoreType.DMA,
            pltpu.SemaphoreType.DMA,
            pltpu.SemaphoreType.DMA,
            pltpu.SemaphoreType.DMA,
            pltpu.SemaphoreType.DMA,
        ],
    )
    def k(curr_h, table_h, scale_h, out_h,
          tok, idx, rows, sv_ref, g0, g1, g2, g3, o0, o1, o2, o3):
        wid = lax.axis_index("s") * _NUM_CORES + lax.axis_index("c")
        base = wid * per_w

        @pl.when(wid == 0)
        def _():
            pltpu.sync_copy(curr_h.at[pl.ds(0, per_w + _PAD)],
                            tok.at[pl.ds(_PAD, per_w + _PAD)])

        @pl.when(wid != 0)
        def _():
            pltpu.sync_copy(curr_h.at[pl.ds(base - _PAD, per_w + 2 * _PAD)],
                            tok)

        pltpu.sync_copy(scale_h, sv_ref)
        sv = sv_ref[...]

        nbuf = 4
        gsem = (g0, g1, g2, g3)
        osem = (o0, o1, o2, o3)
        gather_d = [None] * nchunk
        out_d = [None] * nchunk
        lane = lax.iota(jnp.int32, _L)

        def hash_body(i):
            off = i * _L
            tp = tok[pl.ds(off + _PAD - 1, _L)]
            tc = tok[pl.ds(off + _PAD, _L)]
            h = (jnp.int32(36313) * tc) ^ (jnp.int32(27191) * tp)
            h = h % jnp.int32(_MOD)
            pos = base + off + lane
            h = jnp.where((pos & jnp.int32(seq - 1)) == 0, jnp.int32(_MOD), h)
            c = lax.shift_right_logical(i, 3)
            idx[c, pl.ds((i & (vperchunk - 1)) * _L, _L)] = h

        plsc.parallel_loop(0, vperchunk, unroll=2)(hash_body)
        gather_d[0] = pltpu.async_copy(
            table_h.at[idx.at[0]], rows.at[0], gsem[0])
        plsc.parallel_loop(vperchunk, per_w // _L, unroll=2)(hash_body)

        for c in range(nchunk + 2):
            if 0 < c < nchunk:
                buf = c % nbuf
                if c >= nbuf:
                    out_d[c - nbuf].wait()
                gather_d[c] = pltpu.async_copy(
                    table_h.at[idx.at[c]], rows.at[buf], gsem[buf])
            if c >= 2:
                p = c - 2
                pb = p % nbuf
                gather_d[p].wait()

                @plsc.parallel_loop(0, _CHUNK, unroll=4)
                def _(r, pb=pb):
                    for kk in range(dim // _L):
                        s = pl.ds(kk * _L, _L)
                        rows[pb, r, s] = rows[pb, r, s] * sv

                out_d[p] = pltpu.async_copy(
                    rows.at[pb],
                    out_h.at[pl.ds(base + p * _CHUNK, _CHUNK)],
                    osem[pb])
        for p in range(nchunk - nbuf, nchunk):
            if p >= 0:
                out_d[p].wait()

    return k(curr, table, scale16)


def kernel(token_ids, embed_weight, scale):
    b, seq = token_ids.shape
    dim = embed_weight.shape[1]
    flat = token_ids.reshape(-1).astype(jnp.int32)
    scale16 = jnp.broadcast_to(scale.astype(jnp.float32), (_L,))
    out = _sc_embed(flat, embed_weight, scale16, seq=seq, dim=dim)
    return out.reshape(b, seq, dim)


# speedup vs baseline: 1.3669x; 1.0111x over previous
import functools

import jax
import jax.numpy as jnp
from jax import lax
from jax.experimental import pallas as pl
from jax.experimental.pallas import tpu as pltpu
from jax.experimental.pallas import tpu_sc as plsc

_VOCAB_SIZE = 1000000
_MOD = _VOCAB_SIZE - 1
_L = 16

_NUM_CORES = 2
_NUM_SUBCORES = 16
_NW = _NUM_CORES * _NUM_SUBCORES

_CHUNK = 128
_PAD = 8


@functools.partial(jax.jit, static_argnames=("seq", "dim"))
def _sc_embed(curr, table, scale16, *, seq, dim):
    n = curr.shape[0]
    per_w = n // _NW
    nchunk = per_w // _CHUNK
    vperchunk = _CHUNK // _L

    mesh = plsc.VectorSubcoreMesh(
        core_axis_name="c", subcore_axis_name="s",
        num_cores=_NUM_CORES, num_subcores=_NUM_SUBCORES)

    @functools.partial(
        pl.kernel,
        out_type=jax.ShapeDtypeStruct((n, dim), jnp.float32),
        mesh=mesh,
        scratch_types=[
            pltpu.VMEM((per_w + 2 * _PAD,), jnp.int32),
            pltpu.VMEM((nchunk, _CHUNK), jnp.int32),
            pltpu.VMEM((4, _CHUNK, dim), jnp.float32),
            pltpu.VMEM((_L,), jnp.float32),
            pltpu.SemaphoreType.DMA((4,)),
            pltpu.SemaphoreType.DMA((4,)),
        ],
    )
    def k(curr_h, table_h, scale_h, out_h,
          tok, idx, rows, sv_ref, gsems, osems):
        wid = lax.axis_index("s") * _NUM_CORES + lax.axis_index("c")
        base = wid * per_w

        @pl.when(wid == 0)
        def _():
            pltpu.sync_copy(curr_h.at[pl.ds(0, per_w + _PAD)],
                            tok.at[pl.ds(_PAD, per_w + _PAD)])

        @pl.when(wid != 0)
        def _():
            pltpu.sync_copy(curr_h.at[pl.ds(base - _PAD, per_w + 2 * _PAD)],
                            tok)

        pltpu.sync_copy(scale_h, sv_ref)
        sv = sv_ref[...]

        nbuf = 4
        lane = lax.iota(jnp.int32, _L)

        def hash_body(i):
            off = i * _L
            tp = tok[pl.ds(off + _PAD - 1, _L)]
            tc = tok[pl.ds(off + _PAD, _L)]
            h = (jnp.int32(36313) * tc) ^ (jnp.int32(27191) * tp)
            h = h % jnp.int32(_MOD)
            pos = base + off + lane
            h = jnp.where((pos & jnp.int32(seq - 1)) == 0, jnp.int32(_MOD), h)
            c = lax.shift_right_logical(i, 3)
            idx[c, pl.ds((i & (vperchunk - 1)) * _L, _L)] = h

        plsc.parallel_loop(0, vperchunk, unroll=2)(hash_body)
        pltpu.async_copy(table_h.at[idx.at[0]], rows.at[0], gsems.at[0])
        plsc.parallel_loop(vperchunk, per_w // _L, unroll=2)(hash_body)
        pltpu.async_copy(table_h.at[idx.at[1]], rows.at[1], gsems.at[1])

        def pipe_body(c, _):
            buf = lax.rem(c, nbuf)
            pltpu.make_async_copy(
                table_h.at[idx.at[c]], rows.at[buf], gsems.at[buf]).wait()

            @plsc.parallel_loop(0, _CHUNK, unroll=4)
            def _(r):
                for kk in range(dim // _L):
                    s = pl.ds(kk * _L, _L)
                    rows[buf, r, s] = rows[buf, r, s] * sv

            pltpu.async_copy(
                rows.at[buf],
                out_h.at[pl.ds(base + c * _CHUNK, _CHUNK)],
                osems.at[buf])

            @pl.when(c + 2 < nchunk)
            def _():
                buf2 = lax.rem(c + 2, nbuf)

                @pl.when(c >= 2)
                def _():
                    pltpu.make_async_copy(
                        rows.at[buf2],
                        out_h.at[pl.ds(base + (c - 2) * _CHUNK, _CHUNK)],
                        osems.at[buf2]).wait()

                pltpu.async_copy(
                    table_h.at[idx.at[c + 2]], rows.at[buf2], gsems.at[buf2])

            return 0

        lax.fori_loop(0, nchunk, pipe_body, 0)
        for p in range(nchunk - nbuf, nchunk):
            pb = p % nbuf
            pltpu.make_async_copy(
                rows.at[pb],
                out_h.at[pl.ds(base + p * _CHUNK, _CHUNK)],
                osems.at[pb]).wait()

    return k(curr, table, scale16)


def kernel(token_ids, embed_weight, scale):
    b, seq = token_ids.shape
    dim = embed_weight.shape[1]
    flat = token_ids.reshape(-1).astype(jnp.int32)
    scale16 = jnp.broadcast_to(scale.astype(jnp.float32), (_L,))
    out = _sc_embed(flat, embed_weight, scale16, seq=seq, dim=dim)
    return out.reshape(b, seq, dim)


# speedup vs baseline: 1.4016x; 1.0254x over previous
import functools

import jax
import jax.numpy as jnp
from jax import lax
from jax.experimental import pallas as pl
from jax.experimental.pallas import tpu as pltpu
from jax.experimental.pallas import tpu_sc as plsc

_VOCAB_SIZE = 1000000
_MOD = _VOCAB_SIZE - 1
_L = 16

_NUM_CORES = 2
_NUM_SUBCORES = 16
_NW = _NUM_CORES * _NUM_SUBCORES

_CHUNK = 128
_PAD = 8


@functools.partial(jax.jit, static_argnames=("seq", "dim"))
def _sc_embed(curr, table, scale16, *, seq, dim):
    n = curr.shape[0]
    per_w = n // _NW
    nchunk = per_w // _CHUNK
    vperchunk = _CHUNK // _L

    mesh = plsc.VectorSubcoreMesh(
        core_axis_name="c", subcore_axis_name="s",
        num_cores=_NUM_CORES, num_subcores=_NUM_SUBCORES)

    @functools.partial(
        pl.kernel,
        out_type=jax.ShapeDtypeStruct((n, dim), jnp.float32),
        mesh=mesh,
        scratch_types=[
            pltpu.VMEM((per_w + 2 * _PAD,), jnp.int32),
            pltpu.VMEM((nchunk, _CHUNK), jnp.int32),
            pltpu.VMEM((5, _CHUNK, dim), jnp.float32),
            pltpu.VMEM((_L,), jnp.float32),
            pltpu.SemaphoreType.DMA((5,)),
            pltpu.SemaphoreType.DMA((5,)),
        ],
    )
    def k(curr_h, table_h, scale_h, out_h,
          tok, idx, rows, sv_ref, gsems, osems):
        wid = lax.axis_index("s") * _NUM_CORES + lax.axis_index("c")
        base = wid * per_w

        @pl.when(wid == 0)
        def _():
            pltpu.sync_copy(curr_h.at[pl.ds(0, per_w + _PAD)],
                            tok.at[pl.ds(_PAD, per_w + _PAD)])

        @pl.when(wid != 0)
        def _():
            pltpu.sync_copy(curr_h.at[pl.ds(base - _PAD, per_w + 2 * _PAD)],
                            tok)

        pltpu.sync_copy(scale_h, sv_ref)
        sv = sv_ref[...]

        nbuf = 5
        depth = 3
        lane = lax.iota(jnp.int32, _L)

        def hash_body(i):
            off = i * _L
            tp = tok[pl.ds(off + _PAD - 1, _L)]
            tc = tok[pl.ds(off + _PAD, _L)]
            h = (jnp.int32(36313) * tc) ^ (jnp.int32(27191) * tp)
            h = h % jnp.int32(_MOD)
            pos = base + off + lane
            h = jnp.where((pos & jnp.int32(seq - 1)) == 0, jnp.int32(_MOD), h)
            c = lax.shift_right_logical(i, 3)
            idx[c, pl.ds((i & (vperchunk - 1)) * _L, _L)] = h

        plsc.parallel_loop(0, vperchunk, unroll=2)(hash_body)
        pltpu.async_copy(table_h.at[idx.at[0]], rows.at[0], gsems.at[0])
        plsc.parallel_loop(vperchunk, per_w // _L, unroll=2)(hash_body)
        for j in range(1, 1 + depth):
            pltpu.async_copy(table_h.at[idx.at[j]], rows.at[j], gsems.at[j])

        def pipe_body(c, _):
            buf = lax.rem(c, nbuf)
            pltpu.make_async_copy(
                table_h.at[idx.at[c]], rows.at[buf], gsems.at[buf]).wait()

            @plsc.parallel_loop(0, _CHUNK, unroll=4)
            def _(r):
                for kk in range(dim // _L):
                    s = pl.ds(kk * _L, _L)
                    rows[buf, r, s] = rows[buf, r, s] * sv

            pltpu.async_copy(
                rows.at[buf],
                out_h.at[pl.ds(base + c * _CHUNK, _CHUNK)],
                osems.at[buf])

            nxt = c + 1 + depth
            @pl.when(nxt < nchunk)
            def _():
                buf2 = lax.rem(nxt, nbuf)

                @pl.when(nxt >= nbuf)
                def _():
                    pltpu.make_async_copy(
                        rows.at[buf2],
                        out_h.at[pl.ds(base + (nxt - nbuf) * _CHUNK, _CHUNK)],
                        osems.at[buf2]).wait()

                pltpu.async_copy(
                    table_h.at[idx.at[nxt]], rows.at[buf2], gsems.at[buf2])

            return 0

        lax.fori_loop(0, nchunk, pipe_body, 0)
        for p in range(nchunk - nbuf, nchunk):
            pb = p % nbuf
            pltpu.make_async_copy(
                rows.at[pb],
                out_h.at[pl.ds(base + p * _CHUNK, _CHUNK)],
                osems.at[pb]).wait()

    return k(curr, table, scale16)


def kernel(token_ids, embed_weight, scale):
    b, seq = token_ids.shape
    dim = embed_weight.shape[1]
    flat = token_ids.reshape(-1).astype(jnp.int32)
    scale16 = jnp.broadcast_to(scale.astype(jnp.float32), (_L,))
    out = _sc_embed(flat, embed_weight, scale16, seq=seq, dim=dim)
    return out.reshape(b, seq, dim)


# speedup vs baseline: 1.4232x; 1.0154x over previous
import functools

import jax
import jax.numpy as jnp
from jax import lax
from jax.experimental import pallas as pl
from jax.experimental.pallas import tpu as pltpu
from jax.experimental.pallas import tpu_sc as plsc

_VOCAB_SIZE = 1000000
_MOD = _VOCAB_SIZE - 1
_L = 16

_NUM_CORES = 2
_NUM_SUBCORES = 16
_NW = _NUM_CORES * _NUM_SUBCORES

_CHUNK = 128
_PAD = 8


@functools.partial(jax.jit, static_argnames=("seq", "dim"))
def _sc_embed(curr, table, scale16, *, seq, dim):
    n = curr.shape[0]
    per_w = n // _NW
    nchunk = per_w // _CHUNK
    vperchunk = _CHUNK // _L

    mesh = plsc.VectorSubcoreMesh(
        core_axis_name="c", subcore_axis_name="s",
        num_cores=_NUM_CORES, num_subcores=_NUM_SUBCORES)

    @functools.partial(
        pl.kernel,
        out_type=jax.ShapeDtypeStruct((n, dim), jnp.float32),
        mesh=mesh,
        scratch_types=[
            pltpu.VMEM((per_w + 2 * _PAD,), jnp.int32),
            pltpu.VMEM((nchunk, _CHUNK), jnp.int32),
            pltpu.VMEM((6, _CHUNK, dim), jnp.float32),
            pltpu.VMEM((_L,), jnp.float32),
            pltpu.SemaphoreType.DMA((6,)),
            pltpu.SemaphoreType.DMA((6,)),
        ],
    )
    def k(curr_h, table_h, scale_h, out_h,
          tok, idx, rows, sv_ref, gsems, osems):
        wid = lax.axis_index("s") * _NUM_CORES + lax.axis_index("c")
        base = wid * per_w

        @pl.when(wid == 0)
        def _():
            pltpu.sync_copy(curr_h.at[pl.ds(0, per_w + _PAD)],
                            tok.at[pl.ds(_PAD, per_w + _PAD)])

        @pl.when(wid != 0)
        def _():
            pltpu.sync_copy(curr_h.at[pl.ds(base - _PAD, per_w + 2 * _PAD)],
                            tok)

        pltpu.sync_copy(scale_h, sv_ref)
        sv = sv_ref[...]

        nbuf = 6
        depth = 4
        lane = lax.iota(jnp.int32, _L)

        def hash_body(i):
            off = i * _L
            tp = tok[pl.ds(off + _PAD - 1, _L)]
            tc = tok[pl.ds(off + _PAD, _L)]
            h = (jnp.int32(36313) * tc) ^ (jnp.int32(27191) * tp)
            h = h % jnp.int32(_MOD)
            pos = base + off + lane
            h = jnp.where((pos & jnp.int32(seq - 1)) == 0, jnp.int32(_MOD), h)
            c = lax.shift_right_logical(i, 3)
            idx[c, pl.ds((i & (vperchunk - 1)) * _L, _L)] = h

        plsc.parallel_loop(0, vperchunk, unroll=2)(hash_body)
        pltpu.async_copy(table_h.at[idx.at[0]], rows.at[0], gsems.at[0])
        plsc.parallel_loop(vperchunk, per_w // _L, unroll=2)(hash_body)
        for j in range(1, 1 + depth):
            pltpu.async_copy(table_h.at[idx.at[j]], rows.at[j], gsems.at[j])

        def pipe_body(c, _):
            buf = lax.rem(c, nbuf)
            pltpu.make_async_copy(
                table_h.at[idx.at[c]], rows.at[buf], gsems.at[buf]).wait()

            @plsc.parallel_loop(0, _CHUNK, unroll=4)
            def _(r):
                for kk in range(dim // _L):
                    s = pl.ds(kk * _L, _L)
                    rows[buf, r, s] = rows[buf, r, s] * sv

            pltpu.async_copy(
                rows.at[buf],
                out_h.at[pl.ds(base + c * _CHUNK, _CHUNK)],
                osems.at[buf])

            nxt = c + 1 + depth
            @pl.when(nxt < nchunk)
            def _():
                buf2 = lax.rem(nxt, nbuf)

                @pl.when(nxt >= nbuf)
                def _():
                    pltpu.make_async_copy(
                        rows.at[buf2],
                        out_h.at[pl.ds(base + (nxt - nbuf) * _CHUNK, _CHUNK)],
                        osems.at[buf2]).wait()

                pltpu.async_copy(
                    table_h.at[idx.at[nxt]], rows.at[buf2], gsems.at[buf2])

            return 0

        lax.fori_loop(0, nchunk, pipe_body, 0)
        for p in range(nchunk - nbuf, nchunk):
            pb = p % nbuf
            pltpu.make_async_copy(
                rows.at[pb],
                out_h.at[pl.ds(base + p * _CHUNK, _CHUNK)],
                osems.at[pb]).wait()

    return k(curr, table, scale16)


def kernel(token_ids, embed_weight, scale):
    b, seq = token_ids.shape
    dim = embed_weight.shape[1]
    flat = token_ids.reshape(-1).astype(jnp.int32)
    scale16 = jnp.broadcast_to(scale.astype(jnp.float32), (_L,))
    out = _sc_embed(flat, embed_weight, scale16, seq=seq, dim=dim)
    return out.reshape(b, seq, dim)
